# baseline (speedup 1.0000x reference)
import functools

import jax
import jax.numpy as jnp
from jax import lax
from jax.experimental import pallas as pl
from jax.experimental.pallas import tpu as pltpu
from jax.experimental.pallas import tpu_sc as plsc

N = 10000
E = 320000
NC = 2
NS = 16
NW = NC * NS
EPW = E // NW
CH = 40
CPW = EPW // CH
NPS = 624
NTAIL = N - NPS * NS
NTOFF = NPS * NS

_f32 = jnp.float32
_i32 = jnp.int32


_BN = 1000


def _tc_body(h, W_ref, A_ref, Ar_ref, feat_ref, elr_ref, elrr_ref,
             gA_ref, gB_ref):
    i = pl.program_id(0)
    feat = jnp.dot(h, W_ref[...], preferred_element_type=_f32)
    feat_ref[...] = feat
    elr = jnp.dot(feat, A_ref[...], preferred_element_type=_f32)
    elrr = jnp.dot(feat, Ar_ref[...], preferred_element_type=_f32)
    elr_ref[...] = elr
    elrr_ref[...] = elrr

    @pl.when(i == 0)
    def _():
        gA_ref[...] = jnp.full((8, 16), -1e30, _f32)
        gB_ref[...] = jnp.full((8, 16), -1e30, _f32)

    gA_ref[...] = jnp.maximum(gA_ref[...], jnp.max(elr, axis=0, keepdims=True))
    gB_ref[...] = jnp.maximum(gB_ref[...], jnp.max(elrr, axis=0, keepdims=True))


def _make_tc_first(d_out):
    def body(x_ref, W_ref, A_ref, Ar_ref, feat_ref, elr_ref, elrr_ref,
             gA_ref, gB_ref):
        _tc_body(x_ref[...], W_ref, A_ref, Ar_ref, feat_ref, elr_ref,
                 elrr_ref, gA_ref, gB_ref)

    return pl.pallas_call(
        body,
        grid=(N // _BN,),
        in_specs=[
            pl.BlockSpec((_BN, 128), lambda i: (i, 0)),
            pl.BlockSpec((128, d_out), lambda i: (0, 0)),
            pl.BlockSpec((d_out, 16), lambda i: (0, 0)),
            pl.BlockSpec((d_out, 16), lambda i: (0, 0)),
        ],
        out_specs=[
            pl.BlockSpec((_BN, d_out), lambda i: (i, 0)),
            pl.BlockSpec((_BN, 16), lambda i: (i, 0)),
            pl.BlockSpec((_BN, 16), lambda i: (i, 0)),
            pl.BlockSpec((8, 16), lambda i: (0, 0)),
            pl.BlockSpec((8, 16), lambda i: (0, 0)),
        ],
        out_shape=[
            jax.ShapeDtypeStruct((N, d_out), _f32),
            jax.ShapeDtypeStruct((N, 16), _f32),
            jax.ShapeDtypeStruct((N, 16), _f32),
            jax.ShapeDtypeStruct((8, 16), _f32),
            jax.ShapeDtypeStruct((8, 16), _f32),
        ],
    )


def _make_tc_mid(d_out):
    def body(p0_ref, p1_ref, d0_ref, d1_ref, S_ref, b_ref, W_ref, A_ref,
             Ar_ref, feat_ref, elr_ref, elrr_ref, gA_ref, gB_ref):
        invd = 1.0 / (d0_ref[...] + d1_ref[...] + 1e-9)
        scale = jnp.dot(invd, S_ref[...], preferred_element_type=_f32)
        h = (p0_ref[...] + p1_ref[...]) * scale + b_ref[0:1, :]
        h = jnp.where(h > 0, h, jnp.exp(h) - 1.0)
        _tc_body(h, W_ref, A_ref, Ar_ref, feat_ref, elr_ref, elrr_ref,
                 gA_ref, gB_ref)

    return pl.pallas_call(
        body,
        grid=(N // _BN,),
        in_specs=[
            pl.BlockSpec((_BN, 128), lambda i: (i, 0)),
            pl.BlockSpec((_BN, 128), lambda i: (i, 0)),
            pl.BlockSpec((_BN, 16), lambda i: (i, 0)),
            pl.BlockSpec((_BN, 16), lambda i: (i, 0)),
            pl.BlockSpec((16, 128), lambda i: (0, 0)),
            pl.BlockSpec((8, 128), lambda i: (0, 0)),
            pl.BlockSpec((128, d_out), lambda i: (0, 0)),
            pl.BlockSpec((d_out, 16), lambda i: (0, 0)),
            pl.BlockSpec((d_out, 16), lambda i: (0, 0)),
        ],
        out_specs=[
            pl.BlockSpec((_BN, d_out), lambda i: (i, 0)),
            pl.BlockSpec((_BN, 16), lambda i: (i, 0)),
            pl.BlockSpec((_BN, 16), lambda i: (i, 0)),
            pl.BlockSpec((8, 16), lambda i: (0, 0)),
            pl.BlockSpec((8, 16), lambda i: (0, 0)),
        ],
        out_shape=[
            jax.ShapeDtypeStruct((N, d_out), _f32),
            jax.ShapeDtypeStruct((N, 16), _f32),
            jax.ShapeDtypeStruct((N, 16), _f32),
            jax.ShapeDtypeStruct((8, 16), _f32),
            jax.ShapeDtypeStruct((8, 16), _f32),
        ],
    )


def _make_tc_final():
    def body(p0_ref, p1_ref, d0_ref, d1_ref, S_ref, b_ref, out_ref):
        invd = 1.0 / (d0_ref[...] + d1_ref[...] + 1e-9)
        scale = jnp.dot(invd, S_ref[...], preferred_element_type=_f32)
        out_ref[...] = (p0_ref[...] + p1_ref[...]) * scale + b_ref[0:1, :]

    return pl.pallas_call(
        body,
        grid=(N // _BN,),
        in_specs=[
            pl.BlockSpec((_BN, 40), lambda i: (i, 0)),
            pl.BlockSpec((_BN, 40), lambda i: (i, 0)),
            pl.BlockSpec((_BN, 16), lambda i: (i, 0)),
            pl.BlockSpec((_BN, 16), lambda i: (i, 0)),
            pl.BlockSpec((16, 40), lambda i: (0, 0)),
            pl.BlockSpec((8, 40), lambda i: (0, 0)),
        ],
        out_specs=[pl.BlockSpec((_BN, 40), lambda i: (i, 0))],
        out_shape=[jax.ShapeDtypeStruct((N, 40), _f32)],
    )


_MESH = plsc.VectorSubcoreMesh(core_axis_name="c", subcore_axis_name="s",
                               num_cores=NC, num_subcores=NS)


def _striped_copy(s, src, dst):
    pltpu.sync_copy(src.at[pl.ds(s * NPS, NPS)], dst.at[pl.ds(s * NPS, NPS)])

    @pl.when(s == 0)
    def _():
        pltpu.sync_copy(src.at[pl.ds(NTOFF, NTAIL)],
                        dst.at[pl.ds(NTOFF, NTAIL)])


_NBUF = 5
_NOUT = CPW // _NBUF
_EUR = 8


def _make_scA(H):
    @functools.partial(
        pl.kernel,
        out_type=(
            jax.ShapeDtypeStruct((E, 16), _f32),
            jax.ShapeDtypeStruct((N, 16), _f32),
            jax.ShapeDtypeStruct((N, 16), _f32),
        ),
        mesh=_MESH,
        compiler_params=pltpu.CompilerParams(use_tc_tiling_on_sc=False),
        scratch_types=[
            pltpu.VMEM((_NBUF, CH), _i32),
            pltpu.VMEM((_NBUF, CH), _i32),
            pltpu.VMEM((_NBUF, CH, 16), _f32),
            pltpu.VMEM((_NBUF, CH, 16), _f32),
            pltpu.VMEM((_NBUF, CH, 16), _f32),
            pltpu.VMEM((8, 16), _f32),
            pltpu.VMEM((8, 16), _f32),
            pltpu.VMEM_SHARED((N, 16), _f32),
        ] + [pltpu.SemaphoreType.DMA] * _NBUF,
    )
    def scA(src_h, dst_h, elr_h, elrr_h, gA_h, gB_h, z16_h,
            ee_h, d0_h, d1_h,
            idxs, idxd, ts, td, eeb, ga, gb, den_sh, *sems):
        semG = sems
        c = lax.axis_index("c")
        s = lax.axis_index("s")
        w = s * NC + c
        base = w * EPW
        _striped_copy(s, z16_h, den_sh)
        pltpu.sync_copy(gA_h, ga)
        pltpu.sync_copy(gB_h, gb)
        plsc.subcore_barrier()

        gsum = ga[0] + gb[0]
        g16 = jnp.maximum(gsum, 0.2 * gsum)
        mask = lax.iota(_i32, 16) < H

        def issue_fetch(ci, b):
            off = base + ci * CH
            pltpu.sync_copy(src_h.at[pl.ds(off, CH)], idxs.at[b])
            pltpu.sync_copy(dst_h.at[pl.ds(off, CH)], idxd.at[b])
            pltpu.async_copy(elr_h.at[idxs.at[b]], ts.at[b], semG[b])
            pltpu.async_copy(elrr_h.at[idxd.at[b]], td.at[b], semG[b])

        def wait_fetch(b):
            pltpu.make_async_copy(elr_h.at[idxs.at[b]], ts.at[b],
                                  semG[b]).wait()
            pltpu.make_async_copy(elrr_h.at[idxd.at[b]], td.at[b],
                                  semG[b]).wait()

        def issue_out(ci, b):
            off = base + ci * CH
            pltpu.sync_copy(eeb.at[b], ee_h.at[pl.ds(off, CH)])
            pltpu.sync_copy(eeb.at[b], den_sh.at[idxd.at[b]], add=True)

        issue_fetch(0, 0)
        issue_fetch(1, 1)

        def outer(g, _):
            for b in range(_NBUF):
                ci = g * _NBUF + b
                b2 = (b + 2) % _NBUF

                @pl.when(ci + 2 < CPW)
                def _():
                    issue_fetch(ci + 2, b2)

                wait_fetch(b)
                tsb = ts.at[b]
                tdb = td.at[b]
                eebb = eeb.at[b]

                def edge(eg, _):
                    for k in range(_EUR):
                        e = eg * _EUR + k
                        v = tsb[e] + tdb[e]
                        v = jnp.maximum(v, 0.2 * v)
                        vv = jnp.exp(v - g16)
                        eebb[e] = jnp.where(mask, vv, 0.0)
                    return 0

                lax.fori_loop(0, CH // _EUR, edge, 0)
                issue_out(ci, b)
            return 0

        lax.fori_loop(0, _NOUT, outer, 0)
        plsc.subcore_barrier()

        @pl.when(c == 0)
        def _():
            _striped_copy(s, den_sh, d0_h)

        @pl.when(c == 1)
        def _():
            _striped_copy(s, den_sh, d1_h)

    return scA


def _make_scB(H, Dp):
    NB = Dp // 16

    @functools.partial(
        pl.kernel,
        out_type=(
            jax.ShapeDtypeStruct((N, Dp), _f32),
            jax.ShapeDtypeStruct((N, Dp), _f32),
        ),
        mesh=_MESH,
        compiler_params=pltpu.CompilerParams(use_tc_tiling_on_sc=False),
        scratch_types=[
            pltpu.VMEM((_NBUF, CH), _i32),
            pltpu.VMEM((_NBUF, CH), _i32),
            pltpu.VMEM((_NBUF, CH, Dp), _f32),
            pltpu.VMEM((_NBUF, CH, 16), _f32),
            pltpu.VMEM_SHARED((N, Dp), _f32),
        ] + [pltpu.SemaphoreType.DMA] * _NBUF,
    )
    def scB(src_h, dst_h, feat_h, ee_h, zD_h,
            o0_h, o1_h,
            idxs, idxd, fb, eeb, out_sh, *sems):
        semG = sems
        c = lax.axis_index("c")
        s = lax.axis_index("s")
        w = s * NC + c
        base = w * EPW
        _striped_copy(s, zD_h, out_sh)
        plsc.subcore_barrier()

        def issue_fetch(ci, b):
            off = base + ci * CH
            pltpu.sync_copy(src_h.at[pl.ds(off, CH)], idxs.at[b])
            pltpu.sync_copy(dst_h.at[pl.ds(off, CH)], idxd.at[b])
            pltpu.async_copy(feat_h.at[idxs.at[b]], fb.at[b], semG[b])
            pltpu.async_copy(ee_h.at[pl.ds(off, CH)], eeb.at[b], semG[b])

        def wait_fetch(ci, b):
            off = base + ci * CH
            pltpu.make_async_copy(feat_h.at[idxs.at[b]], fb.at[b],
                                  semG[b]).wait()
            pltpu.make_async_copy(ee_h.at[pl.ds(off, CH)], eeb.at[b],
                                  semG[b]).wait()

        def issue_out(b):
            pltpu.sync_copy(fb.at[b], out_sh.at[idxd.at[b]], add=True)

        issue_fetch(0, 0)
        issue_fetch(1, 1)

        def outer(g, _):
            for b in range(_NBUF):
                ci = g * _NBUF + b
                b2 = (b + 2) % _NBUF

                @pl.when(ci + 2 < CPW)
                def _():
                    issue_fetch(ci + 2, b2)

                wait_fetch(ci, b)
                fbb = fb.at[b]
                eebb = eeb.at[b]

                def edge(eg, _):
                    for k in range(_EUR):
                        e = eg * _EUR + k
                        alpha = eebb[e]
                        for j in range(NB):
                            hh = j if H > 1 else 0
                            a = alpha[hh]
                            fbb[e, pl.ds(16 * j, 16)] = (
                                fbb[e, pl.ds(16 * j, 16)] * a)
                    return 0

                lax.fori_loop(0, CH // _EUR, edge, 0)
                issue_out(b)
            return 0

        lax.fori_loop(0, _NOUT, outer, 0)
        plsc.subcore_barrier()

        @pl.when(c == 0)
        def _():
            _striped_copy(s, out_sh, o0_h)

        @pl.when(c == 1)
        def _():
            _striped_copy(s, out_sh, o1_h)

    return scB


_tc0 = _make_tc_first(128)
_tc1 = _make_tc_mid(128)
_tc2 = _make_tc_mid(48)
_tcf = _make_tc_final()
_scA8 = _make_scA(8)
_scA1 = _make_scA(1)
_scB8 = _make_scB(8, 128)
_scB1 = _make_scB(1, 48)


def _build_A(al, ar, H, HID, Dp):
    D = H * HID
    rows = jnp.arange(D)
    heads = rows // HID
    A = jnp.zeros((Dp, 16), _f32)
    A = A.at[rows, heads].set(al.reshape(-1))
    A = A.at[rows, H + heads].set(ar.reshape(-1))
    return A


def kernel(x, edge_index, W0, al0, ar0, b0, W1, al1, ar1, b1,
           W2, al2, ar2, b2):
    src = edge_index[0]
    dst = edge_index[1]

    A0 = _build_A(al0, ar0, 8, 16, 128)
    A0r = _build_A(ar0, al0, 8, 16, 128)
    A1 = _build_A(al1, ar1, 8, 16, 128)
    A1r = _build_A(ar1, al1, 8, 16, 128)
    A2 = _build_A(al2, ar2, 1, 40, 48)
    A2r = _build_A(ar2, al2, 1, 40, 48)
    W2p = jnp.pad(W2, ((0, 0), (0, 8)))

    z16 = jnp.zeros((N, 16), _f32)
    z128 = jnp.zeros((N, 128), _f32)
    z48 = jnp.zeros((N, 48), _f32)
    b0b = jnp.broadcast_to(b0[None, :], (8, 128))
    b1b = jnp.broadcast_to(b1[None, :], (8, 128))
    b2b = jnp.broadcast_to(b2[None, :], (8, 40))

    cols = jnp.arange(128)
    S8 = (jnp.arange(16)[:, None] == cols[None, :] // 16).astype(_f32)
    S1 = (jnp.arange(16)[:, None] == 0).astype(_f32) * jnp.ones((1, 40), _f32)

    feat0, elr0, elr0r, gA0, gB0 = _tc0(x, W0, A0, A0r)
    ee0, d00, d01 = _scA8(src, dst, elr0, elr0r, gA0, gB0, z16)
    o00, o01 = _scB8(src, dst, feat0, ee0, z128)

    feat1, elr1, elr1r, gA1, gB1 = _tc1(o00, o01, d00, d01, S8, b0b,
                                        W1, A1, A1r)
    ee1, d10, d11 = _scA8(src, dst, elr1, elr1r, gA1, gB1, z16)
    o10, o11 = _scB8(src, dst, feat1, ee1, z128)

    feat2, elr2, elr2r, gA2, gB2 = _tc2(o10, o11, d10, d11, S8, b1b,
                                        W2p, A2, A2r)
    ee2, d20, d21 = _scA1(src, dst, elr2, elr2r, gA2, gB2, z16)
    o20, o21 = _scB1(src, dst, feat2, ee2, z48)

    (logits,) = _tcf(o20[:, :40], o21[:, :40], d20, d21, S1, b2b)
    return logits


# speedup vs baseline: 1.8381x; 1.8381x over previous
import functools

import jax
import jax.numpy as jnp
from jax import lax
from jax.experimental import pallas as pl
from jax.experimental.pallas import tpu as pltpu
from jax.experimental.pallas import tpu_sc as plsc

N = 10000
E = 320000
NC = 2
NS = 16
NW = NC * NS
EPW = E // NW
CH = 40
CPW = EPW // CH
NPS = 624
NTAIL = N - NPS * NS
NTOFF = NPS * NS

_f32 = jnp.float32
_i32 = jnp.int32


_BN = 1000


def _tc_body(h, W_ref, A_ref, Ar_ref, feat_ref, elr_ref, elrr_ref,
             gA_ref, gB_ref):
    i = pl.program_id(0)
    feat = jnp.dot(h, W_ref[...], preferred_element_type=_f32)
    feat_ref[...] = feat
    elr = jnp.dot(feat, A_ref[...], preferred_element_type=_f32)
    elrr = jnp.dot(feat, Ar_ref[...], preferred_element_type=_f32)
    elr_ref[...] = elr
    elrr_ref[...] = elrr

    @pl.when(i == 0)
    def _():
        gA_ref[...] = jnp.full((8, 16), -1e30, _f32)
        gB_ref[...] = jnp.full((8, 16), -1e30, _f32)

    gA_ref[...] = jnp.maximum(gA_ref[...], jnp.max(elr, axis=0, keepdims=True))
    gB_ref[...] = jnp.maximum(gB_ref[...], jnp.max(elrr, axis=0, keepdims=True))


def _make_tc_first(d_out):
    def body(x_ref, W_ref, A_ref, Ar_ref, feat_ref, elr_ref, elrr_ref,
             gA_ref, gB_ref):
        _tc_body(x_ref[...], W_ref, A_ref, Ar_ref, feat_ref, elr_ref,
                 elrr_ref, gA_ref, gB_ref)

    return pl.pallas_call(
        body,
        grid=(N // _BN,),
        in_specs=[
            pl.BlockSpec((_BN, 128), lambda i: (i, 0)),
            pl.BlockSpec((128, d_out), lambda i: (0, 0)),
            pl.BlockSpec((d_out, 16), lambda i: (0, 0)),
            pl.BlockSpec((d_out, 16), lambda i: (0, 0)),
        ],
        out_specs=[
            pl.BlockSpec((_BN, d_out), lambda i: (i, 0)),
            pl.BlockSpec((_BN, 16), lambda i: (i, 0)),
            pl.BlockSpec((_BN, 16), lambda i: (i, 0)),
            pl.BlockSpec((8, 16), lambda i: (0, 0)),
            pl.BlockSpec((8, 16), lambda i: (0, 0)),
        ],
        out_shape=[
            jax.ShapeDtypeStruct((N, d_out), _f32),
            jax.ShapeDtypeStruct((N, 16), _f32),
            jax.ShapeDtypeStruct((N, 16), _f32),
            jax.ShapeDtypeStruct((8, 16), _f32),
            jax.ShapeDtypeStruct((8, 16), _f32),
        ],
    )


def _make_tc_mid(d_out):
    def body(p0_ref, p1_ref, d0_ref, d1_ref, S_ref, b_ref, W_ref, A_ref,
             Ar_ref, feat_ref, elr_ref, elrr_ref, gA_ref, gB_ref):
        invd = 1.0 / (d0_ref[...] + d1_ref[...] + 1e-9)
        scale = jnp.dot(invd, S_ref[...], preferred_element_type=_f32)
        h = (p0_ref[...] + p1_ref[...]) * scale + b_ref[0:1, :]
        h = jnp.where(h > 0, h, jnp.exp(h) - 1.0)
        _tc_body(h, W_ref, A_ref, Ar_ref, feat_ref, elr_ref, elrr_ref,
                 gA_ref, gB_ref)

    return pl.pallas_call(
        body,
        grid=(N // _BN,),
        in_specs=[
            pl.BlockSpec((_BN, 128), lambda i: (i, 0)),
            pl.BlockSpec((_BN, 128), lambda i: (i, 0)),
            pl.BlockSpec((_BN, 16), lambda i: (i, 0)),
            pl.BlockSpec((_BN, 16), lambda i: (i, 0)),
            pl.BlockSpec((16, 128), lambda i: (0, 0)),
            pl.BlockSpec((8, 128), lambda i: (0, 0)),
            pl.BlockSpec((128, d_out), lambda i: (0, 0)),
            pl.BlockSpec((d_out, 16), lambda i: (0, 0)),
            pl.BlockSpec((d_out, 16), lambda i: (0, 0)),
        ],
        out_specs=[
            pl.BlockSpec((_BN, d_out), lambda i: (i, 0)),
            pl.BlockSpec((_BN, 16), lambda i: (i, 0)),
            pl.BlockSpec((_BN, 16), lambda i: (i, 0)),
            pl.BlockSpec((8, 16), lambda i: (0, 0)),
            pl.BlockSpec((8, 16), lambda i: (0, 0)),
        ],
        out_shape=[
            jax.ShapeDtypeStruct((N, d_out), _f32),
            jax.ShapeDtypeStruct((N, 16), _f32),
            jax.ShapeDtypeStruct((N, 16), _f32),
            jax.ShapeDtypeStruct((8, 16), _f32),
            jax.ShapeDtypeStruct((8, 16), _f32),
        ],
    )


def _make_tc_final():
    def body(p0_ref, p1_ref, d0_ref, d1_ref, S_ref, b_ref, out_ref):
        invd = 1.0 / (d0_ref[...] + d1_ref[...] + 1e-9)
        scale = jnp.dot(invd, S_ref[...], preferred_element_type=_f32)
        out_ref[...] = (p0_ref[...] + p1_ref[...]) * scale + b_ref[0:1, :]

    return pl.pallas_call(
        body,
        grid=(N // _BN,),
        in_specs=[
            pl.BlockSpec((_BN, 40), lambda i: (i, 0)),
            pl.BlockSpec((_BN, 40), lambda i: (i, 0)),
            pl.BlockSpec((_BN, 16), lambda i: (i, 0)),
            pl.BlockSpec((_BN, 16), lambda i: (i, 0)),
            pl.BlockSpec((16, 40), lambda i: (0, 0)),
            pl.BlockSpec((8, 40), lambda i: (0, 0)),
        ],
        out_specs=[pl.BlockSpec((_BN, 40), lambda i: (i, 0))],
        out_shape=[jax.ShapeDtypeStruct((N, 40), _f32)],
    )


_MESH = plsc.VectorSubcoreMesh(core_axis_name="c", subcore_axis_name="s",
                               num_cores=NC, num_subcores=NS)


def _striped_copy(s, src, dst):
    pltpu.sync_copy(src.at[pl.ds(s * NPS, NPS)], dst.at[pl.ds(s * NPS, NPS)])

    @pl.when(s == 0)
    def _():
        pltpu.sync_copy(src.at[pl.ds(NTOFF, NTAIL)],
                        dst.at[pl.ds(NTOFF, NTAIL)])


_NBUF = 5
_NOUT = CPW // _NBUF
_EUR = 8


def _make_scA(H):
    @functools.partial(
        pl.kernel,
        out_type=(
            jax.ShapeDtypeStruct((E, 16), _f32),
            jax.ShapeDtypeStruct((N, 16), _f32),
            jax.ShapeDtypeStruct((N, 16), _f32),
        ),
        mesh=_MESH,
        compiler_params=pltpu.CompilerParams(use_tc_tiling_on_sc=False),
        scratch_types=[
            pltpu.VMEM((_NBUF, 2, CH), _i32),
            pltpu.VMEM((_NBUF, CH, 16), _f32),
            pltpu.VMEM((_NBUF, CH, 16), _f32),
            pltpu.VMEM((_NBUF, CH, 16), _f32),
            pltpu.VMEM((8, 16), _f32),
            pltpu.VMEM((8, 16), _f32),
            pltpu.VMEM_SHARED((N, 16), _f32),
        ] + [pltpu.SemaphoreType.DMA] * (2 * _NBUF),
    )
    def scA(e2_h, elr_h, elrr_h, gA_h, gB_h, z16_h,
            ee_h, d0_h, d1_h,
            idxb, ts, td, eeb, ga, gb, den_sh, *sems):
        semG = sems[:_NBUF]
        semI = sems[_NBUF:]
        c = lax.axis_index("c")
        s = lax.axis_index("s")
        w = s * NC + c
        base = w * EPW
        _striped_copy(s, z16_h, den_sh)
        pltpu.sync_copy(gA_h, ga)
        pltpu.sync_copy(gB_h, gb)
        plsc.subcore_barrier()

        gsum = ga[0] + gb[0]
        g16 = jnp.maximum(gsum, 0.2 * gsum)
        mask = lax.iota(_i32, 16) < H

        def issue_idx(ci, b):
            off = base + ci * CH
            pltpu.async_copy(e2_h.at[:, pl.ds(off, CH)], idxb.at[b],
                             semI[b])

        def wait_idx(b):
            pltpu.make_async_copy(e2_h.at[:, pl.ds(0, CH)], idxb.at[b],
                                  semI[b]).wait()

        def issue_gather(b):
            pltpu.async_copy(elr_h.at[idxb.at[b, 0]], ts.at[b], semG[b])
            pltpu.async_copy(elrr_h.at[idxb.at[b, 1]], td.at[b], semG[b])

        def wait_gather(b):
            pltpu.make_async_copy(elr_h.at[idxb.at[b, 0]], ts.at[b],
                                  semG[b]).wait()
            pltpu.make_async_copy(elrr_h.at[idxb.at[b, 1]], td.at[b],
                                  semG[b]).wait()

        def issue_out(ci, b):
            off = base + ci * CH
            pltpu.sync_copy(eeb.at[b], ee_h.at[pl.ds(off, CH)])
            pltpu.sync_copy(eeb.at[b], den_sh.at[idxb.at[b, 1]], add=True)

        issue_idx(0, 0)
        wait_idx(0)
        issue_gather(0)
        issue_idx(1, 1)

        def outer(g, _):
            for b in range(_NBUF):
                ci = g * _NBUF + b
                b1 = (b + 1) % _NBUF
                b2 = (b + 2) % _NBUF

                @pl.when(ci + 1 < CPW)
                def _():
                    wait_idx(b1)
                    issue_gather(b1)

                @pl.when(ci + 2 < CPW)
                def _():
                    issue_idx(ci + 2, b2)

                wait_gather(b)
                tsb = ts.at[b]
                tdb = td.at[b]
                eebb = eeb.at[b]

                def edge(eg, _):
                    for k in range(_EUR):
                        e = eg * _EUR + k
                        v = tsb[e] + tdb[e]
                        v = jnp.maximum(v, 0.2 * v)
                        vv = jnp.exp(v - g16)
                        eebb[e] = jnp.where(mask, vv, 0.0)
                    return 0

                lax.fori_loop(0, CH // _EUR, edge, 0)
                issue_out(ci, b)
            return 0

        lax.fori_loop(0, _NOUT, outer, 0)
        plsc.subcore_barrier()

        @pl.when(c == 0)
        def _():
            _striped_copy(s, den_sh, d0_h)

        @pl.when(c == 1)
        def _():
            _striped_copy(s, den_sh, d1_h)

    return scA


def _make_scB(H, Dp):
    NB = Dp // 16

    @functools.partial(
        pl.kernel,
        out_type=(
            jax.ShapeDtypeStruct((N, Dp), _f32),
            jax.ShapeDtypeStruct((N, Dp), _f32),
        ),
        mesh=_MESH,
        compiler_params=pltpu.CompilerParams(use_tc_tiling_on_sc=False),
        scratch_types=[
            pltpu.VMEM((_NBUF, 2, CH), _i32),
            pltpu.VMEM((_NBUF, CH, Dp), _f32),
            pltpu.VMEM((_NBUF, CH, 16), _f32),
            pltpu.VMEM_SHARED((N, Dp), _f32),
        ] + [pltpu.SemaphoreType.DMA] * (2 * _NBUF),
    )
    def scB(e2_h, feat_h, ee_h, zD_h,
            o0_h, o1_h,
            idxb, fb, eeb, out_sh, *sems):
        semG = sems[:_NBUF]
        semI = sems[_NBUF:]
        c = lax.axis_index("c")
        s = lax.axis_index("s")
        w = s * NC + c
        base = w * EPW
        _striped_copy(s, zD_h, out_sh)
        plsc.subcore_barrier()

        def issue_idx(ci, b):
            off = base + ci * CH
            pltpu.async_copy(e2_h.at[:, pl.ds(off, CH)], idxb.at[b],
                             semI[b])

        def wait_idx(b):
            pltpu.make_async_copy(e2_h.at[:, pl.ds(0, CH)], idxb.at[b],
                                  semI[b]).wait()

        def issue_gather(ci, b):
            off = base + ci * CH
            pltpu.async_copy(feat_h.at[idxb.at[b, 0]], fb.at[b], semG[b])
            pltpu.async_copy(ee_h.at[pl.ds(off, CH)], eeb.at[b], semG[b])

        def wait_gather(ci, b):
            off = base + ci * CH
            pltpu.make_async_copy(feat_h.at[idxb.at[b, 0]], fb.at[b],
                                  semG[b]).wait()
            pltpu.make_async_copy(ee_h.at[pl.ds(off, CH)], eeb.at[b],
                                  semG[b]).wait()

        def issue_out(b):
            pltpu.sync_copy(fb.at[b], out_sh.at[idxb.at[b, 1]], add=True)

        issue_idx(0, 0)
        wait_idx(0)
        issue_gather(0, 0)
        issue_idx(1, 1)

        def outer(g, _):
            for b in range(_NBUF):
                ci = g * _NBUF + b
                b1 = (b + 1) % _NBUF
                b2 = (b + 2) % _NBUF

                @pl.when(ci + 1 < CPW)
                def _():
                    wait_idx(b1)
                    issue_gather(ci + 1, b1)

                @pl.when(ci + 2 < CPW)
                def _():
                    issue_idx(ci + 2, b2)

                wait_gather(ci, b)
                fbb = fb.at[b]
                eebb = eeb.at[b]

                def edge(eg, _):
                    for k in range(_EUR):
                        e = eg * _EUR + k
                        alpha = eebb[e]
                        for j in range(NB):
                            hh = j if H > 1 else 0
                            a = alpha[hh]
                            fbb[e, pl.ds(16 * j, 16)] = (
                                fbb[e, pl.ds(16 * j, 16)] * a)
                    return 0

                lax.fori_loop(0, CH // _EUR, edge, 0)
                issue_out(b)
            return 0

        lax.fori_loop(0, _NOUT, outer, 0)
        plsc.subcore_barrier()

        @pl.when(c == 0)
        def _():
            _striped_copy(s, out_sh, o0_h)

        @pl.when(c == 1)
        def _():
            _striped_copy(s, out_sh, o1_h)

    return scB


_tc0 = _make_tc_first(128)
_tc1 = _make_tc_mid(128)
_tc2 = _make_tc_mid(48)
_tcf = _make_tc_final()
_scA8 = _make_scA(8)
_scA1 = _make_scA(1)
_scB8 = _make_scB(8, 128)
_scB1 = _make_scB(1, 48)


def _build_A(al, ar, H, HID, Dp):
    D = H * HID
    rows = jnp.arange(D)
    heads = rows // HID
    A = jnp.zeros((Dp, 16), _f32)
    A = A.at[rows, heads].set(al.reshape(-1))
    A = A.at[rows, H + heads].set(ar.reshape(-1))
    return A


def kernel(x, edge_index, W0, al0, ar0, b0, W1, al1, ar1, b1,
           W2, al2, ar2, b2):
    e2 = edge_index.astype(_i32)

    A0 = _build_A(al0, ar0, 8, 16, 128)
    A0r = _build_A(ar0, al0, 8, 16, 128)
    A1 = _build_A(al1, ar1, 8, 16, 128)
    A1r = _build_A(ar1, al1, 8, 16, 128)
    A2 = _build_A(al2, ar2, 1, 40, 48)
    A2r = _build_A(ar2, al2, 1, 40, 48)
    W2p = jnp.pad(W2, ((0, 0), (0, 8)))

    z16 = jnp.zeros((N, 16), _f32)
    z128 = jnp.zeros((N, 128), _f32)
    z48 = jnp.zeros((N, 48), _f32)
    b0b = jnp.broadcast_to(b0[None, :], (8, 128))
    b1b = jnp.broadcast_to(b1[None, :], (8, 128))
    b2b = jnp.broadcast_to(b2[None, :], (8, 40))

    cols = jnp.arange(128)
    S8 = (jnp.arange(16)[:, None] == cols[None, :] // 16).astype(_f32)
    S1 = (jnp.arange(16)[:, None] == 0).astype(_f32) * jnp.ones((1, 40), _f32)

    feat0, elr0, elr0r, gA0, gB0 = _tc0(x, W0, A0, A0r)
    ee0, d00, d01 = _scA8(e2, elr0, elr0r, gA0, gB0, z16)
    o00, o01 = _scB8(e2, feat0, ee0, z128)

    feat1, elr1, elr1r, gA1, gB1 = _tc1(o00, o01, d00, d01, S8, b0b,
                                        W1, A1, A1r)
    ee1, d10, d11 = _scA8(e2, elr1, elr1r, gA1, gB1, z16)
    o10, o11 = _scB8(e2, feat1, ee1, z128)

    feat2, elr2, elr2r, gA2, gB2 = _tc2(o10, o11, d10, d11, S8, b1b,
                                        W2p, A2, A2r)
    ee2, d20, d21 = _scA1(e2, elr2, elr2r, gA2, gB2, z16)
    o20, o21 = _scB1(e2, feat2, ee2, z48)

    (logits,) = _tcf(o20[:, :40], o21[:, :40], d20, d21, S1, b2b)
    return logits
